# baseline (speedup 1.0000x reference)
import functools

import jax
import jax.numpy as jnp
from jax import lax
from jax.experimental import pallas as pl
from jax.experimental.pallas import tpu as pltpu
from jax.experimental.pallas import tpu_sc as plsc

D = 128
V0, V1, V2 = 5, 6, 2
VT = V0 * V1 * V2

NC, NS = 2, 16
NW = NC * NS

C = 128
KF = 2
CB = C * KF
NSLOT = 2


def _table_body(w0_ref, w1_ref, w2_ref, t_ref):
    for r in range(VT):
        i0, i1, i2 = r // (V1 * V2), (r // V2) % V1, r % V2
        t_ref[pl.ds(r, 1), :] = (
            w0_ref[pl.ds(i0, 1), :]
            + w1_ref[pl.ds(i1, 1), :]
            + w2_ref[pl.ds(i2, 1), :]
        )


def _build_table(W0, W1, W2):
    return pl.pallas_call(
        _table_body,
        out_shape=jax.ShapeDtypeStruct((VT, D), jnp.float32),
    )(W0, W1, W2)


def _sc_body(bpw, tab_hbm, ea0_hbm, ea1_hbm, ea2_hbm, out_hbm, ea0_v, ea1_v,
             ea2_v, idx_v, rows_v, gsem, wsem):
    wid = lax.axis_index("s") * NC + lax.axis_index("c")
    base = wid * bpw
    pltpu.sync_copy(ea0_hbm.at[pl.ds(base, bpw)], ea0_v)
    pltpu.sync_copy(ea1_hbm.at[pl.ds(base, bpw)], ea1_v)
    pltpu.sync_copy(ea2_hbm.at[pl.ds(base, bpw)], ea2_v)

    def idx_body(j, carry):
        i0 = ea0_v[pl.ds(j * 16, 16)]
        i1 = ea1_v[pl.ds(j * 16, 16)]
        i2 = ea2_v[pl.ds(j * 16, 16)]
        cidx = i0 * (V1 * V2) + i1 * V2 + i2
        idx_v[j // (C // 16), pl.ds((j % (C // 16)) * 16, 16)] = cidx
        return carry

    n_grp = bpw // 16
    lax.fori_loop(0, n_grp, idx_body, 0)

    n_blk = bpw // CB
    tail = bpw - n_blk * CB
    zero16 = jnp.zeros((16,), jnp.int32)
    if tail:
        for g in range(tail // 16, C // 16):
            idx_v[n_grp // (C // 16), pl.ds(g * 16, 16)] = zero16

    def blk_body(o, carry):
        s = lax.rem(o, NSLOT) * CB
        cps = []
        for f in range(KF):
            cps.append(
                pltpu.async_copy(
                    tab_hbm.at[idx_v.at[o * KF + f]],
                    rows_v.at[pl.ds(s + f * C, C)],
                    gsem,
                )
            )
        for cp in cps:
            cp.wait()

        @pl.when(o > 0)
        def _():
            pltpu.make_async_copy(
                rows_v.at[pl.ds(lax.rem(o + 1, NSLOT) * CB, CB)],
                out_hbm.at[pl.ds(base + (o - 1) * CB, CB)],
                wsem,
            ).wait()

        pltpu.async_copy(
            rows_v.at[pl.ds(s, CB)],
            out_hbm.at[pl.ds(base + o * CB, CB)],
            wsem,
        )
        return carry

    lax.fori_loop(0, n_blk, blk_body, 0)
    pltpu.make_async_copy(
        rows_v.at[pl.ds(0, CB)],
        out_hbm.at[pl.ds(base + (n_blk - 1) * CB, CB)],
        wsem,
    ).wait()

    if tail:
        pltpu.async_copy(
            tab_hbm.at[idx_v.at[n_blk * KF]],
            rows_v.at[pl.ds(0, C)],
            gsem,
        ).wait()
        pltpu.sync_copy(
            rows_v.at[pl.ds(0, tail)],
            out_hbm.at[pl.ds(base + n_blk * CB, tail)],
        )


def kernel(edge_attr, W0, W1, W2):
    E = edge_attr.shape[0]
    assert E % (NW * 16) == 0
    bpw = E // NW
    assert bpw % 16 == 0 and (bpw % CB) % 16 == 0

    table = _build_table(W0, W1, W2)
    ea = edge_attr.astype(jnp.int32)
    ea0, ea1, ea2 = ea[:, 0], ea[:, 1], ea[:, 2]

    mesh = plsc.VectorSubcoreMesh(core_axis_name="c", subcore_axis_name="s")
    sc_kernel = functools.partial(
        pl.kernel,
        out_type=jax.ShapeDtypeStruct((E, D), jnp.float32),
        mesh=mesh,
        scratch_types=[
            pltpu.VMEM((bpw,), jnp.int32),
            pltpu.VMEM((bpw,), jnp.int32),
            pltpu.VMEM((bpw,), jnp.int32),
            pltpu.VMEM((pl.cdiv(bpw, C), C), jnp.int32),
            pltpu.VMEM((NSLOT * CB, D), jnp.float32),
            pltpu.SemaphoreType.DMA,
            pltpu.SemaphoreType.DMA,
        ],
    )(functools.partial(_sc_body, bpw))
    return sc_kernel(table, ea0, ea1, ea2)


# speedup vs baseline: 1.0008x; 1.0008x over previous
import functools

import jax
import jax.numpy as jnp
from jax import lax
from jax.experimental import pallas as pl
from jax.experimental.pallas import tpu as pltpu
from jax.experimental.pallas import tpu_sc as plsc

D = 128
V0, V1, V2 = 5, 6, 2
VT = V0 * V1 * V2

NC, NS = 2, 16
NW = NC * NS

C = 128
NSLOT = 6
DG = 3


def _table_body(w0_ref, w1_ref, w2_ref, t_ref):
    for r in range(VT):
        i0, i1, i2 = r // (V1 * V2), (r // V2) % V1, r % V2
        t_ref[pl.ds(r, 1), :] = (
            w0_ref[pl.ds(i0, 1), :]
            + w1_ref[pl.ds(i1, 1), :]
            + w2_ref[pl.ds(i2, 1), :]
        )


def _build_table(W0, W1, W2):
    return pl.pallas_call(
        _table_body,
        out_shape=jax.ShapeDtypeStruct((VT, D), jnp.float32),
    )(W0, W1, W2)


def _sc_body(bpw, tab_hbm, ea0_hbm, ea1_hbm, ea2_hbm, out_hbm, ea0_v, ea1_v,
             ea2_v, idx_v, rows_v, *sems):
    gsem, wsem = sems[:NSLOT], sems[NSLOT:]
    wid = lax.axis_index("s") * NC + lax.axis_index("c")
    base = wid * bpw

    n_grp = bpw // 16
    eh = ((bpw + 31) // 32) * 16
    s2 = bpw - eh
    g2 = s2 // 16

    def make_idx_body(grp0):
        def idx_body(j, carry):
            o = (j - grp0) * 16
            i0 = ea0_v[pl.ds(o, 16)]
            i1 = ea1_v[pl.ds(o, 16)]
            i2 = ea2_v[pl.ds(o, 16)]
            cidx = i0 * (V1 * V2) + i1 * V2 + i2
            idx_v[j // (C // 16), pl.ds((j % (C // 16)) * 16, 16)] = cidx
            return carry
        return idx_body

    pltpu.sync_copy(ea0_hbm.at[pl.ds(base, eh)], ea0_v)
    pltpu.sync_copy(ea1_hbm.at[pl.ds(base, eh)], ea1_v)
    pltpu.sync_copy(ea2_hbm.at[pl.ds(base, eh)], ea2_v)
    lax.fori_loop(0, eh // 16, make_idx_body(0), 0)
    pltpu.sync_copy(ea0_hbm.at[pl.ds(base + s2, eh)], ea0_v)
    pltpu.sync_copy(ea1_hbm.at[pl.ds(base + s2, eh)], ea1_v)
    pltpu.sync_copy(ea2_hbm.at[pl.ds(base + s2, eh)], ea2_v)
    lax.fori_loop(g2, n_grp, make_idx_body(g2), 0)

    n_blk = bpw // C
    tail = bpw - n_blk * C
    if tail:
        zero16 = jnp.zeros((16,), jnp.int32)
        for g in range(tail // 16, C // 16):
            idx_v[n_grp // (C // 16), pl.ds(g * 16, 16)] = zero16

    def fire_gather(k):
        pltpu.async_copy(
            tab_hbm.at[idx_v.at[k]],
            rows_v.at[pl.ds((k % NSLOT) * C, C)],
            gsem[k % NSLOT],
        )

    def wait_write(o):
        pltpu.make_async_copy(
            rows_v.at[pl.ds((o % NSLOT) * C, C)],
            out_hbm.at[pl.ds(base + o * C, C)],
            wsem[o % NSLOT],
        ).wait()

    for t in range(n_blk + DG):
        if t < n_blk:
            if t >= NSLOT:
                wait_write(t - NSLOT)
            fire_gather(t)
        if t >= DG:
            o = t - DG
            s = o % NSLOT
            pltpu.make_async_copy(
                tab_hbm.at[idx_v.at[o]],
                rows_v.at[pl.ds(s * C, C)],
                gsem[s],
            ).wait()
            pltpu.async_copy(
                rows_v.at[pl.ds(s * C, C)],
                out_hbm.at[pl.ds(base + o * C, C)],
                wsem[s],
            )
    for o in range(max(0, n_blk - NSLOT), n_blk):
        wait_write(o)

    if tail:
        pltpu.async_copy(
            tab_hbm.at[idx_v.at[n_blk]],
            rows_v.at[pl.ds(0, C)],
            gsem[0],
        ).wait()
        pltpu.sync_copy(
            rows_v.at[pl.ds(0, tail)],
            out_hbm.at[pl.ds(base + n_blk * C, tail)],
        )


def kernel(edge_attr, W0, W1, W2):
    E = edge_attr.shape[0]
    bpw = E // NW
    assert E == bpw * NW and bpw % 16 == 0 and (bpw - ((bpw + 31) // 32) * 16) % 8 == 0

    table = _build_table(W0, W1, W2)
    ea = edge_attr.astype(jnp.int32)
    ea0, ea1, ea2 = ea[:, 0], ea[:, 1], ea[:, 2]

    eh = ((bpw + 31) // 32) * 16
    mesh = plsc.VectorSubcoreMesh(core_axis_name="c", subcore_axis_name="s")
    sc_kernel = functools.partial(
        pl.kernel,
        out_type=jax.ShapeDtypeStruct((E, D), jnp.float32),
        mesh=mesh,
        scratch_types=[
            pltpu.VMEM((eh,), jnp.int32),
            pltpu.VMEM((eh,), jnp.int32),
            pltpu.VMEM((eh,), jnp.int32),
            pltpu.VMEM((pl.cdiv(bpw, C), C), jnp.int32),
            pltpu.VMEM((NSLOT * C, D), jnp.float32),
        ] + [pltpu.SemaphoreType.DMA] * (2 * NSLOT),
    )(functools.partial(_sc_body, bpw))
    return sc_kernel(table, ea0, ea1, ea2)


# speedup vs baseline: 4.6994x; 4.6955x over previous
import functools

import jax
import jax.numpy as jnp
from jax import lax
from jax.experimental import pallas as pl
from jax.experimental.pallas import tpu as pltpu
from jax.experimental.pallas import tpu_sc as plsc

D = 128
V0, V1, V2 = 5, 6, 2
VT = V0 * V1 * V2
VTP = 64

NC, NS = 2, 16
NW = NC * NS

C = 128
NSLOT = 2


def _table_body(w0_ref, w1_ref, w2_ref, t_ref):
    for r in range(VTP):
        q = min(r, VT - 1)
        i0, i1, i2 = q // (V1 * V2), (q // V2) % V1, q % V2
        t_ref[pl.ds(r, 1), :] = (
            w0_ref[pl.ds(i0, 1), :]
            + w1_ref[pl.ds(i1, 1), :]
            + w2_ref[pl.ds(i2, 1), :]
        )


def _build_table(W0, W1, W2):
    return pl.pallas_call(
        _table_body,
        out_shape=jax.ShapeDtypeStruct((VTP, D), jnp.float32),
    )(W0, W1, W2)


def _expand_block(t_v, idx_v, blk, rows_v, slot, n):
    for g in range(n // 16):
        civ = idx_v[blk, pl.ds(g * 16, 16)] * D
        for l in range(16):
            ci = civ[l]
            i = g * 16 + l
            for c in range(D // 16):
                rows_v[slot * C + i, pl.ds(c * 16, 16)] = (
                    t_v[pl.ds(ci + c * 16, 16)])


def _sc_body(bpw, tab_hbm, ea0_hbm, ea1_hbm, ea2_hbm, out_hbm, t_v, ea0_v,
             ea1_v, ea2_v, idx_v, rows_v, wsem0, wsem1):
    wid = lax.axis_index("s") * NC + lax.axis_index("c")
    base = wid * bpw

    pltpu.sync_copy(tab_hbm, t_v)

    n_grp = bpw // 16
    eh = ((bpw + 31) // 32) * 16
    s2 = bpw - eh
    g2 = s2 // 16

    def make_idx_body(grp0):
        def idx_body(j, carry):
            o = (j - grp0) * 16
            i0 = ea0_v[pl.ds(o, 16)]
            i1 = ea1_v[pl.ds(o, 16)]
            i2 = ea2_v[pl.ds(o, 16)]
            cidx = i0 * (V1 * V2) + i1 * V2 + i2
            idx_v[j // (C // 16), pl.ds((j % (C // 16)) * 16, 16)] = cidx
            return carry
        return idx_body

    pltpu.sync_copy(ea0_hbm.at[pl.ds(base, eh)], ea0_v)
    pltpu.sync_copy(ea1_hbm.at[pl.ds(base, eh)], ea1_v)
    pltpu.sync_copy(ea2_hbm.at[pl.ds(base, eh)], ea2_v)
    lax.fori_loop(0, eh // 16, make_idx_body(0), 0)
    pltpu.sync_copy(ea0_hbm.at[pl.ds(base + s2, eh)], ea0_v)
    pltpu.sync_copy(ea1_hbm.at[pl.ds(base + s2, eh)], ea1_v)
    pltpu.sync_copy(ea2_hbm.at[pl.ds(base + s2, eh)], ea2_v)
    lax.fori_loop(g2, n_grp, make_idx_body(g2), 0)

    n_blk = bpw // C
    tail = bpw - n_blk * C
    assert n_blk % NSLOT == 0

    def wait_write(sem, slot, o):
        pltpu.make_async_copy(
            rows_v.at[pl.ds(slot * C, C)],
            out_hbm.at[pl.ds(base + lax.max(o, 0) * C, C)],
            sem,
        ).wait()

    def pair_body(p, carry):
        for slot, sem in ((0, wsem0), (1, wsem1)):
            o = p * NSLOT + slot

            @pl.when(p > 0)
            def _():
                wait_write(sem, slot, o - NSLOT)

            _expand_block(t_v, idx_v, o, rows_v, slot, C)
            pltpu.async_copy(
                rows_v.at[pl.ds(slot * C, C)],
                out_hbm.at[pl.ds(base + o * C, C)],
                sem,
            )
        return carry

    lax.fori_loop(0, n_blk // NSLOT, pair_body, 0)
    wait_write(wsem0, 0, n_blk - NSLOT)
    wait_write(wsem1, 1, n_blk - NSLOT + 1)

    if tail:
        _expand_block(t_v, idx_v, n_blk, rows_v, 0, tail)
        pltpu.sync_copy(
            rows_v.at[pl.ds(0, tail)],
            out_hbm.at[pl.ds(base + n_blk * C, tail)],
        )


def kernel(edge_attr, W0, W1, W2):
    E = edge_attr.shape[0]
    bpw = E // NW
    assert E == bpw * NW and bpw % 16 == 0
    assert (bpw - ((bpw + 31) // 32) * 16) % 8 == 0

    table = _build_table(W0, W1, W2)
    ea = edge_attr.astype(jnp.int32)
    ea0, ea1, ea2 = ea[:, 0], ea[:, 1], ea[:, 2]

    eh = ((bpw + 31) // 32) * 16
    mesh = plsc.VectorSubcoreMesh(core_axis_name="c", subcore_axis_name="s")
    sc_kernel = functools.partial(
        pl.kernel,
        out_type=jax.ShapeDtypeStruct((E, D), jnp.float32),
        mesh=mesh,
        scratch_types=[
            pltpu.VMEM((VTP * D,), jnp.float32),
            pltpu.VMEM((eh,), jnp.int32),
            pltpu.VMEM((eh,), jnp.int32),
            pltpu.VMEM((eh,), jnp.int32),
            pltpu.VMEM((pl.cdiv(bpw, C), C), jnp.int32),
            pltpu.VMEM((NSLOT * C, D), jnp.float32),
            pltpu.SemaphoreType.DMA,
            pltpu.SemaphoreType.DMA,
        ],
    )(functools.partial(_sc_body, bpw))
    return sc_kernel(table.reshape(-1), ea0, ea1, ea2)
